# baseline (speedup 1.0000x reference)
import dataclasses
import functools

import jax
import jax.numpy as jnp
from jax import lax
from jax.experimental import pallas as pl
from jax.experimental.pallas import tpu as pltpu
from jax.experimental.pallas import tpu_sc as plsc

N = 10000
E = 320000
D = 128

NC = 2
NS = 16
LANES = 16
NW = NC * NS

DH = D // 2
NPAD = 10240
B = 128
NB_TILE = 160
EPAD = NS * NB_TILE * B
ROWS_PER_TILE = NPAD // NS

_HIGHEST = jax.lax.Precision.HIGHEST


def _dot(a, b):
  return jax.lax.dot(a, b, precision=_HIGHEST,
                     preferred_element_type=jnp.float32)


def _lrelu(v):
  return jnp.where(v >= 0, v, v * jnp.float32(0.2))


BLK = 2048
GRID = NPAD // BLK

_row_spec = pl.BlockSpec((BLK, D), lambda i: (i, 0))
_col_spec = pl.BlockSpec((BLK, 1), lambda i: (i, 0))
_xs_spec = pl.BlockSpec((NC, BLK, DH), lambda i: (0, i, 0))
_w_spec = pl.BlockSpec((D, D), lambda i: (0, 0))
_v_spec = pl.BlockSpec((1, D), lambda i: (0, 0))


def _prep_body(x_ref, ws_ref, wd_ref, avs_ref, avd_ref, wl_ref, bl_ref,
               xs_ref, asrc_ref, ad_ref, skip_ref):
  x = x_ref[...]
  xs = _dot(x, ws_ref[...])
  xd = _dot(x, wd_ref[...])
  xs_ref[0] = xs[:, :DH]
  xs_ref[1] = xs[:, DH:]
  asrc_ref[...] = jnp.sum(xs * avs_ref[...], axis=1, keepdims=True)
  ad_ref[...] = jnp.sum(xd * avd_ref[...], axis=1, keepdims=True)
  skip_ref[...] = _dot(x, wl_ref[...]) + bl_ref[...]


def _tc_prep(xp, Ws, Wd, avs, avd, Wl, bl):
  out_shape = (
      jax.ShapeDtypeStruct((NC, NPAD, DH), jnp.float32),
      jax.ShapeDtypeStruct((NPAD, 1), jnp.float32),
      jax.ShapeDtypeStruct((NPAD, 1), jnp.float32),
      jax.ShapeDtypeStruct((NPAD, D), jnp.float32),
  )
  return pl.pallas_call(
      _prep_body,
      grid=(GRID,),
      in_specs=[_row_spec, _w_spec, _w_spec, _v_spec, _v_spec, _w_spec,
                _v_spec],
      out_specs=(_xs_spec, _col_spec, _col_spec, _row_spec),
      out_shape=out_shape,
  )(xp, Ws, Wd, avs, avd, Wl, bl)


def _m_body(asrc_ref, m_ref):
  m_ref[...] = jnp.full((1, D), jnp.max(asrc_ref[...]), jnp.float32)


def _tc_m(asrc):
  m_row = pl.pallas_call(
      _m_body,
      out_shape=jax.ShapeDtypeStruct((1, D), jnp.float32),
  )(asrc)
  return m_row.reshape(D)[:LANES]


def _gat_h(n_ref, d_ref, b_ref, skip_ref):
  numer = jnp.concatenate([n_ref[0], n_ref[1]], axis=1)
  return numer / (d_ref[...] + jnp.float32(1e-16)) + b_ref[...] + skip_ref[...]


def _mid_body(n_ref, d_ref, b1_ref, skip1_ref, ws_ref, wd_ref, avs_ref,
              avd_ref, wl_ref, bl_ref,
              xs_ref, asrc_ref, ad_ref, skip_ref):
  h = jnp.maximum(_gat_h(n_ref, d_ref, b1_ref, skip1_ref), 0.0)
  base = pl.program_id(0) * BLK
  rowid = base + jax.lax.broadcasted_iota(jnp.int32, (BLK, 1), 0)
  h = jnp.where(rowid < N, h, 0.0)
  xs = _dot(h, ws_ref[...])
  xd = _dot(h, wd_ref[...])
  xs_ref[0] = xs[:, :DH]
  xs_ref[1] = xs[:, DH:]
  asrc_ref[...] = jnp.sum(xs * avs_ref[...], axis=1, keepdims=True)
  ad_ref[...] = jnp.sum(xd * avd_ref[...], axis=1, keepdims=True)
  skip_ref[...] = _dot(h, wl_ref[...]) + bl_ref[...]


def _tc_mid(numer, denom, b1, skip1, Ws, Wd, avs, avd, Wl, bl):
  out_shape = (
      jax.ShapeDtypeStruct((NC, NPAD, DH), jnp.float32),
      jax.ShapeDtypeStruct((NPAD, 1), jnp.float32),
      jax.ShapeDtypeStruct((NPAD, 1), jnp.float32),
      jax.ShapeDtypeStruct((NPAD, D), jnp.float32),
  )
  return pl.pallas_call(
      _mid_body,
      grid=(GRID,),
      in_specs=[_xs_spec, _col_spec, _v_spec, _row_spec, _w_spec, _w_spec,
                _v_spec, _v_spec, _w_spec, _v_spec],
      out_specs=(_xs_spec, _col_spec, _col_spec, _row_spec),
      out_shape=out_shape,
  )(numer, denom, b1, skip1, Ws, Wd, avs, avd, Wl, bl)


def _final_body(n_ref, d_ref, b2_ref, skip2_ref, out_ref):
  out_ref[...] = _gat_h(n_ref, d_ref, b2_ref, skip2_ref)


def _tc_final(numer, denom, b2, skip2):
  return pl.pallas_call(
      _final_body,
      grid=(GRID,),
      in_specs=[_xs_spec, _col_spec, _v_spec, _row_spec],
      out_specs=_row_spec,
      out_shape=jax.ShapeDtypeStruct((NPAD, D), jnp.float32),
  )(numer, denom, b2, skip2)


NBUF = 4
GSTEP = 2 * NBUF


@functools.cache
def _make_sc_edge_kernel():
  mesh = plsc.VectorSubcoreMesh(
      core_axis_name="c", subcore_axis_name="s",
      num_cores=NC, num_subcores=NS)

  cp = pltpu.CompilerParams()
  if "needs_layout_passes" in pltpu.CompilerParams.__dataclass_fields__:
    cp = dataclasses.replace(cp, needs_layout_passes=False)
  if "use_tc_tiling_on_sc" in pltpu.CompilerParams.__dataclass_fields__:
    cp = dataclasses.replace(cp, use_tc_tiling_on_sc=False)

  scratch = [
      pltpu.VMEM((NPAD,), jnp.float32),
      pltpu.VMEM((NPAD,), jnp.float32),
      pltpu.VMEM((LANES,), jnp.float32),
  ]
  scratch += [pltpu.VMEM((NBUF, B), jnp.int32)] * 2
  scratch += [pltpu.VMEM((NBUF, B), jnp.int32)] * 2
  scratch += [pltpu.VMEM((B, DH), jnp.float32)] * NBUF
  scratch += [pltpu.VMEM((B,), jnp.float32)] * NBUF
  scratch += [
      pltpu.VMEM_SHARED((NPAD, DH), jnp.float32),
      pltpu.VMEM_SHARED((NPAD,), jnp.float32),
  ]
  scratch += [pltpu.SemaphoreType.DMA] * (3 * NBUF + 4)

  @functools.partial(
      pl.kernel,
      compiler_params=cp,
      out_type=(
          jax.ShapeDtypeStruct((NC, NPAD, DH), jnp.float32),
          jax.ShapeDtypeStruct((NC, NPAD), jnp.float32),
      ),
      mesh=mesh,
      scratch_types=scratch,
  )
  def _sc_edge_kernel(xs_hbm, asrc_hbm, ad_hbm, m_hbm, src_hbm, dst_hbm,
                      zr_hbm, zv_hbm, numer_hbm, denom_hbm,
                      tas_v, tad_v, tm_v, *rest):
    idxs_sl = rest[0:2]
    idxd_sl = rest[2:4]
    rows_bf = rest[4:4 + NBUF]
    ex_bf = rest[4 + NBUF:4 + 2 * NBUF]
    sh_numer, sh_denom = rest[4 + 2 * NBUF:6 + 2 * NBUF]
    sems = rest[6 + 2 * NBUF:]
    gsem = sems[0:NBUF]
    srow = sems[NBUF:2 * NBUF]
    sden = sems[2 * NBUF:3 * NBUF]
    rsem = sems[3 * NBUF:3 * NBUF + 4]
    _sc_edge_body(xs_hbm, asrc_hbm, ad_hbm, m_hbm, src_hbm, dst_hbm,
                  zr_hbm, zv_hbm, numer_hbm, denom_hbm,
                  tas_v, tad_v, tm_v, idxs_sl, idxd_sl, rows_bf, ex_bf,
                  sh_numer, sh_denom, gsem, srow, sden, rsem)

  return _sc_edge_kernel


def _sc_edge_body(xs_hbm, asrc_hbm, ad_hbm, m_hbm, src_hbm, dst_hbm,
                  zr_hbm, zv_hbm, numer_hbm, denom_hbm,
                  tas_v, tad_v, tm_v, idxs_sl, idxd_sl, rows_bf, ex_bf,
                  sh_numer, sh_denom, gsem, srow, sden, rsem):
  c = lax.axis_index("c")
  s = lax.axis_index("s")
  r0 = s * ROWS_PER_TILE
  xs_half = xs_hbm.at[c]
  rbase = s * NB_TILE

  pltpu.sync_copy(zr_hbm, sh_numer.at[pl.ds(r0, ROWS_PER_TILE)])
  pltpu.sync_copy(zv_hbm, sh_denom.at[pl.ds(r0, ROWS_PER_TILE)])

  pltpu.sync_copy(asrc_hbm, tas_v)
  pltpu.sync_copy(ad_hbm, tad_v)
  pltpu.sync_copy(m_hbm, tm_v)
  pltpu.sync_copy(src_hbm.at[pl.ds(rbase, NBUF)], idxs_sl[0].at[...])
  pltpu.sync_copy(dst_hbm.at[pl.ds(rbase, NBUF)], idxd_sl[0].at[...])
  plsc.subcore_barrier()

  mv = tm_v[...]

  def _ex_compute(sl, i, exv):
    for k in range(B // LANES):
      ds16 = pl.ds(k * LANES, LANES)
      sv = idxs_sl[sl][i, ds16]
      dv = idxd_sl[sl][i, ds16]
      ad_g = plsc.load_gather(tad_v, [dv])
      a = plsc.load_gather(tas_v, [sv]) + ad_g
      exv[ds16] = jnp.exp(_lrelu(a) - _lrelu(mv + ad_g))

  def _scale(rowsv, exv):
    @pl.loop(0, B, step=LANES)
    def _grp(r16):
      ex16 = exv[pl.ds(r16, LANES)]
      for blk in range(0, LANES, 4):
        evs = [
            ex16.at[jnp.full((LANES,), blk + t, jnp.int32)].get(
                mode="promise_in_bounds")
            for t in range(4)
        ]
        rr = [r16 + blk + t for t in range(4)]
        loads = [[rowsv[rr[t], pl.ds(j * LANES, LANES)]
                  for j in range(DH // LANES)] for t in range(4)]
        for t in range(4):
          for j in range(DH // LANES):
            rowsv[rr[t], pl.ds(j * LANES, LANES)] = loads[t][j] * evs[t]

  HB = B // 2

  def _start_gather(sl, i, b):
    idx_row = idxs_sl[sl].at[i]
    pltpu.async_copy(xs_half.at[idx_row.at[pl.ds(0, HB)]],
                     rows_bf[b].at[pl.ds(0, HB)], gsem[b])
    pltpu.async_copy(xs_half.at[idx_row.at[pl.ds(HB, HB)]],
                     rows_bf[b].at[pl.ds(HB, HB)], gsem[b])

  def _wait_gather(sl, i, b):
    idx_row = idxs_sl[0].at[0]
    pltpu.make_async_copy(xs_half.at[idx_row.at[pl.ds(0, HB)]],
                          rows_bf[b].at[pl.ds(0, HB)], gsem[b]).wait()
    pltpu.make_async_copy(xs_half.at[idx_row.at[pl.ds(HB, HB)]],
                          rows_bf[b].at[pl.ds(HB, HB)], gsem[b]).wait()

  def _start_scatter(sl, i, b):
    pltpu.async_copy(rows_bf[b], sh_numer.at[idxd_sl[sl].at[i]], srow[b],
                     add=True)
    pltpu.async_copy(ex_bf[b], sh_denom.at[idxd_sl[sl].at[i]], sden[b],
                     add=True)

  def _wait_scatter(b):
    pltpu.make_async_copy(rows_bf[b], sh_numer.at[idxd_sl[0].at[0]],
                          srow[b]).wait()
    pltpu.make_async_copy(ex_bf[b], sh_denom.at[idxd_sl[0].at[0]],
                          sden[b]).wait()

  def _start_refill(sl, g_next):
    off = rbase + g_next
    pltpu.async_copy(src_hbm.at[pl.ds(off, NBUF)], idxs_sl[sl].at[...],
                     rsem[2 * sl])
    pltpu.async_copy(dst_hbm.at[pl.ds(off, NBUF)], idxd_sl[sl].at[...],
                     rsem[2 * sl + 1])

  def _wait_refill(sl):
    pltpu.make_async_copy(src_hbm.at[pl.ds(0, NBUF)], idxs_sl[sl].at[...],
                          rsem[2 * sl]).wait()
    pltpu.make_async_copy(dst_hbm.at[pl.ds(0, NBUF)], idxd_sl[sl].at[...],
                          rsem[2 * sl + 1]).wait()

  _start_gather(0, 0, 0)
  _start_gather(0, 1, 1)

  @pl.loop(0, NB_TILE, step=GSTEP)
  def _batch(g):
    for half in range(2):
      sl = half
      so = 1 - half
      gb = g + half * NBUF
      for i in range(NBUF):
        b = i
        _ex_compute(sl, i, ex_bf[b])
        _wait_gather(sl, i, b)
        n2 = (i + 2) % NBUF
        if half == 0 and i < 2:
          @pl.when(g > 0)
          def _():
            _wait_scatter(n2)
        else:
          _wait_scatter(n2)
        if i == 1:
          if half == 0:
            _start_refill(so, gb + NBUF)
          else:
            @pl.when(g + GSTEP < NB_TILE)
            def _():
              _start_refill(so, gb + NBUF)
        if i < 2:
          _start_gather(sl, i + 2, n2)
        elif i == 2:
          if half == 0:
            _wait_refill(so)
            _start_gather(so, 0, n2)
          else:
            @pl.when(g + GSTEP < NB_TILE)
            def _():
              _wait_refill(so)
              _start_gather(so, 0, n2)
        else:
          if half == 0:
            _start_gather(so, 1, n2)
          else:
            @pl.when(g + GSTEP < NB_TILE)
            def _():
              _start_gather(so, 1, n2)
        _scale(rows_bf[b], ex_bf[b])
        _start_scatter(sl, i, b)

  for i in range(2, NBUF):
    _wait_scatter(i)

  plsc.subcore_barrier()
  pltpu.sync_copy(sh_numer.at[pl.ds(r0, ROWS_PER_TILE)],
                  numer_hbm.at[c, pl.ds(r0, ROWS_PER_TILE)])
  pltpu.sync_copy(sh_denom.at[pl.ds(r0, ROWS_PER_TILE)],
                  denom_hbm.at[c, pl.ds(r0, ROWS_PER_TILE)])


def kernel(x, edge_index, W1s, W1d, a1s, a1d, b1, Wl1, bl1,
           W2s, W2d, a2s, a2d, b2, Wl2, bl2):
  src = edge_index[0].astype(jnp.int32)
  dst = edge_index[1].astype(jnp.int32)
  pad = jnp.full((EPAD - E,), N, jnp.int32)
  srcp = jnp.concatenate([src, pad]).reshape(NS * NB_TILE, B)
  dstp = jnp.concatenate([dst, pad]).reshape(NS * NB_TILE, B)

  xp = jnp.zeros((NPAD, D), jnp.float32).at[:N].set(x)
  zr = jnp.zeros((ROWS_PER_TILE, DH), jnp.float32)
  zv = jnp.zeros((ROWS_PER_TILE,), jnp.float32)

  a1s_v = a1s.reshape(1, D)
  a1d_v = a1d.reshape(1, D)
  a2s_v = a2s.reshape(1, D)
  a2d_v = a2d.reshape(1, D)

  sc_edge = _make_sc_edge_kernel()

  xs1, asrc1, ad1, skip1 = _tc_prep(
      xp, W1s, W1d, a1s_v, a1d_v, Wl1, bl1.reshape(1, D))
  m1 = _tc_m(asrc1)
  numer1, denom1 = sc_edge(
      xs1, asrc1.reshape(NPAD), ad1.reshape(NPAD), m1,
      srcp, dstp, zr, zv)

  xs2, asrc2, ad2, skip2 = _tc_mid(
      numer1, denom1[0].reshape(NPAD, 1), b1.reshape(1, D), skip1,
      W2s, W2d, a2s_v, a2d_v, Wl2, bl2.reshape(1, D))
  m2 = _tc_m(asrc2)
  numer2, denom2 = sc_edge(
      xs2, asrc2.reshape(NPAD), ad2.reshape(NPAD), m2,
      srcp, dstp, zr, zv)

  out = _tc_final(numer2, denom2[0].reshape(NPAD, 1), b2.reshape(1, D),
                  skip2)
  return out[:N]


# speedup vs baseline: 1.4885x; 1.4885x over previous
import dataclasses
import functools

import jax
import jax.numpy as jnp
from jax import lax
from jax.experimental import pallas as pl
from jax.experimental.pallas import tpu as pltpu
from jax.experimental.pallas import tpu_sc as plsc

N = 10000
E = 320000
D = 128

NC = 2
NS = 16
LANES = 16
NW = NC * NS

DH = D // 2
NPAD = 10240
B = 96
NB_TILE = 216
EPAD = NS * NB_TILE * B
ROWS_PER_TILE = NPAD // NS

_HIGHEST = jax.lax.Precision.HIGHEST


def _dot(a, b):
  return jax.lax.dot(a, b, precision=_HIGHEST,
                     preferred_element_type=jnp.float32)


def _lrelu(v):
  return jnp.where(v >= 0, v, v * jnp.float32(0.2))


BLK = 2048
GRID = NPAD // BLK

_row_spec = pl.BlockSpec((BLK, D), lambda i: (i, 0))
_col_spec = pl.BlockSpec((BLK, 1), lambda i: (i, 0))
_xs_spec = pl.BlockSpec((NC, BLK, DH), lambda i: (0, i, 0))
_w_spec = pl.BlockSpec((D, D), lambda i: (0, 0))
_v_spec = pl.BlockSpec((1, D), lambda i: (0, 0))


def _prep_body(x_ref, ws_ref, wd_ref, avs_ref, avd_ref, wl_ref, bl_ref,
               xs_ref, asrc_ref, ad_ref, skip_ref):
  x = x_ref[...]
  xs = _dot(x, ws_ref[...])
  xd = _dot(x, wd_ref[...])
  xs_ref[0] = xs[:, :DH]
  xs_ref[1] = xs[:, DH:]
  asrc_ref[...] = jnp.sum(xs * avs_ref[...], axis=1, keepdims=True)
  ad_ref[...] = jnp.sum(xd * avd_ref[...], axis=1, keepdims=True)
  skip_ref[...] = _dot(x, wl_ref[...]) + bl_ref[...]


def _tc_prep(xp, Ws, Wd, avs, avd, Wl, bl):
  out_shape = (
      jax.ShapeDtypeStruct((NC, NPAD, DH), jnp.float32),
      jax.ShapeDtypeStruct((NPAD, 1), jnp.float32),
      jax.ShapeDtypeStruct((NPAD, 1), jnp.float32),
      jax.ShapeDtypeStruct((NPAD, D), jnp.float32),
  )
  return pl.pallas_call(
      _prep_body,
      grid=(GRID,),
      in_specs=[_row_spec, _w_spec, _w_spec, _v_spec, _v_spec, _w_spec,
                _v_spec],
      out_specs=(_xs_spec, _col_spec, _col_spec, _row_spec),
      out_shape=out_shape,
  )(xp, Ws, Wd, avs, avd, Wl, bl)


def _m_body(asrc_ref, m_ref):
  m_ref[...] = jnp.full((1, D), jnp.max(asrc_ref[...]), jnp.float32)


def _tc_m(asrc):
  m_row = pl.pallas_call(
      _m_body,
      out_shape=jax.ShapeDtypeStruct((1, D), jnp.float32),
  )(asrc)
  return m_row.reshape(D)[:LANES]


def _gat_h(n_ref, d_ref, b_ref, skip_ref):
  numer = jnp.concatenate([n_ref[0], n_ref[1]], axis=1)
  return numer / (d_ref[...] + jnp.float32(1e-16)) + b_ref[...] + skip_ref[...]


def _mid_body(n_ref, d_ref, b1_ref, skip1_ref, ws_ref, wd_ref, avs_ref,
              avd_ref, wl_ref, bl_ref,
              xs_ref, asrc_ref, ad_ref, skip_ref):
  h = jnp.maximum(_gat_h(n_ref, d_ref, b1_ref, skip1_ref), 0.0)
  base = pl.program_id(0) * BLK
  rowid = base + jax.lax.broadcasted_iota(jnp.int32, (BLK, 1), 0)
  h = jnp.where(rowid < N, h, 0.0)
  xs = _dot(h, ws_ref[...])
  xd = _dot(h, wd_ref[...])
  xs_ref[0] = xs[:, :DH]
  xs_ref[1] = xs[:, DH:]
  asrc_ref[...] = jnp.sum(xs * avs_ref[...], axis=1, keepdims=True)
  ad_ref[...] = jnp.sum(xd * avd_ref[...], axis=1, keepdims=True)
  skip_ref[...] = _dot(h, wl_ref[...]) + bl_ref[...]


def _tc_mid(numer, denom, b1, skip1, Ws, Wd, avs, avd, Wl, bl):
  out_shape = (
      jax.ShapeDtypeStruct((NC, NPAD, DH), jnp.float32),
      jax.ShapeDtypeStruct((NPAD, 1), jnp.float32),
      jax.ShapeDtypeStruct((NPAD, 1), jnp.float32),
      jax.ShapeDtypeStruct((NPAD, D), jnp.float32),
  )
  return pl.pallas_call(
      _mid_body,
      grid=(GRID,),
      in_specs=[_xs_spec, _col_spec, _v_spec, _row_spec, _w_spec, _w_spec,
                _v_spec, _v_spec, _w_spec, _v_spec],
      out_specs=(_xs_spec, _col_spec, _col_spec, _row_spec),
      out_shape=out_shape,
  )(numer, denom, b1, skip1, Ws, Wd, avs, avd, Wl, bl)


def _final_body(n_ref, d_ref, b2_ref, skip2_ref, out_ref):
  out_ref[...] = _gat_h(n_ref, d_ref, b2_ref, skip2_ref)


def _tc_final(numer, denom, b2, skip2):
  return pl.pallas_call(
      _final_body,
      grid=(GRID,),
      in_specs=[_xs_spec, _col_spec, _v_spec, _row_spec],
      out_specs=_row_spec,
      out_shape=jax.ShapeDtypeStruct((NPAD, D), jnp.float32),
  )(numer, denom, b2, skip2)


NBUF = 4
GSTEP = 2 * NBUF


@functools.cache
def _make_sc_edge_kernel():
  mesh = plsc.VectorSubcoreMesh(
      core_axis_name="c", subcore_axis_name="s",
      num_cores=NC, num_subcores=NS)

  cp = pltpu.CompilerParams()
  if "needs_layout_passes" in pltpu.CompilerParams.__dataclass_fields__:
    cp = dataclasses.replace(cp, needs_layout_passes=False)
  if "use_tc_tiling_on_sc" in pltpu.CompilerParams.__dataclass_fields__:
    cp = dataclasses.replace(cp, use_tc_tiling_on_sc=False)

  scratch = [
      pltpu.VMEM((NPAD,), jnp.float32),
      pltpu.VMEM((NPAD,), jnp.float32),
      pltpu.VMEM((LANES,), jnp.float32),
  ]
  scratch += [pltpu.VMEM((NBUF, B), jnp.int32)] * 2
  scratch += [pltpu.VMEM((NBUF, B), jnp.int32)] * 2
  scratch += [pltpu.VMEM((B, DH), jnp.float32)] * NBUF
  scratch += [pltpu.VMEM((B,), jnp.float32)] * NBUF
  scratch += [
      pltpu.VMEM_SHARED((NPAD, DH), jnp.float32),
      pltpu.VMEM_SHARED((NPAD, DH), jnp.float32),
      pltpu.VMEM_SHARED((NPAD,), jnp.float32),
  ]
  scratch += [pltpu.SemaphoreType.DMA] * (3 * NBUF + 4)

  @functools.partial(
      pl.kernel,
      compiler_params=cp,
      out_type=(
          jax.ShapeDtypeStruct((NC, NPAD, DH), jnp.float32),
          jax.ShapeDtypeStruct((NC, NPAD), jnp.float32),
      ),
      mesh=mesh,
      scratch_types=scratch,
  )
  def _sc_edge_kernel(xs_hbm, asrc_hbm, ad_hbm, m_hbm, src_hbm, dst_hbm,
                      zr_hbm, zv_hbm, numer_hbm, denom_hbm,
                      tas_v, tad_v, tm_v, *rest):
    idxs_sl = rest[0:2]
    idxd_sl = rest[2:4]
    rows_bf = rest[4:4 + NBUF]
    ex_bf = rest[4 + NBUF:4 + 2 * NBUF]
    xs_sh, sh_numer, sh_denom = rest[4 + 2 * NBUF:7 + 2 * NBUF]
    sems = rest[7 + 2 * NBUF:]
    gsem = sems[0:NBUF]
    srow = sems[NBUF:2 * NBUF]
    sden = sems[2 * NBUF:3 * NBUF]
    rsem = sems[3 * NBUF:3 * NBUF + 4]
    _sc_edge_body(xs_hbm, asrc_hbm, ad_hbm, m_hbm, src_hbm, dst_hbm,
                  zr_hbm, zv_hbm, numer_hbm, denom_hbm,
                  tas_v, tad_v, tm_v, idxs_sl, idxd_sl, rows_bf, ex_bf,
                  xs_sh, sh_numer, sh_denom, gsem, srow, sden, rsem)

  return _sc_edge_kernel


def _sc_edge_body(xs_hbm, asrc_hbm, ad_hbm, m_hbm, src_hbm, dst_hbm,
                  zr_hbm, zv_hbm, numer_hbm, denom_hbm,
                  tas_v, tad_v, tm_v, idxs_sl, idxd_sl, rows_bf, ex_bf,
                  xs_sh, sh_numer, sh_denom, gsem, srow, sden, rsem):
  c = lax.axis_index("c")
  s = lax.axis_index("s")
  r0 = s * ROWS_PER_TILE
  xs_half = xs_hbm.at[c]
  rbase = s * NB_TILE

  pltpu.sync_copy(zr_hbm, sh_numer.at[pl.ds(r0, ROWS_PER_TILE)])
  pltpu.sync_copy(zv_hbm, sh_denom.at[pl.ds(r0, ROWS_PER_TILE)])
  pltpu.sync_copy(xs_half.at[pl.ds(r0, ROWS_PER_TILE)],
                  xs_sh.at[pl.ds(r0, ROWS_PER_TILE)])

  pltpu.sync_copy(asrc_hbm, tas_v)
  pltpu.sync_copy(ad_hbm, tad_v)
  pltpu.sync_copy(m_hbm, tm_v)
  pltpu.sync_copy(src_hbm.at[pl.ds(rbase, NBUF)], idxs_sl[0].at[...])
  pltpu.sync_copy(dst_hbm.at[pl.ds(rbase, NBUF)], idxd_sl[0].at[...])
  plsc.subcore_barrier()

  mv = tm_v[...]

  def _ex_compute(sl, i, exv):
    for k in range(B // LANES):
      ds16 = pl.ds(k * LANES, LANES)
      sv = idxs_sl[sl][i, ds16]
      dv = idxd_sl[sl][i, ds16]
      ad_g = plsc.load_gather(tad_v, [dv])
      a = plsc.load_gather(tas_v, [sv]) + ad_g
      exv[ds16] = jnp.exp(_lrelu(a) - _lrelu(mv + ad_g))

  def _scale(rowsv, exv):
    @pl.loop(0, B, step=LANES)
    def _grp(r16):
      ex16 = exv[pl.ds(r16, LANES)]
      for blk in range(0, LANES, 4):
        evs = [
            ex16.at[jnp.full((LANES,), blk + t, jnp.int32)].get(
                mode="promise_in_bounds")
            for t in range(4)
        ]
        rr = [r16 + blk + t for t in range(4)]
        loads = [[rowsv[rr[t], pl.ds(j * LANES, LANES)]
                  for j in range(DH // LANES)] for t in range(4)]
        for t in range(4):
          for j in range(DH // LANES):
            rowsv[rr[t], pl.ds(j * LANES, LANES)] = loads[t][j] * evs[t]

  def _start_gather(sl, i, b):
    pltpu.async_copy(xs_sh.at[idxs_sl[sl].at[i]], rows_bf[b], gsem[b])

  def _wait_gather(sl, i, b):
    pltpu.make_async_copy(xs_sh.at[idxs_sl[sl].at[i]], rows_bf[b],
                          gsem[b]).wait()

  def _start_scatter(sl, i, b):
    pltpu.async_copy(rows_bf[b], sh_numer.at[idxd_sl[sl].at[i]], srow[b],
                     add=True)
    pltpu.async_copy(ex_bf[b], sh_denom.at[idxd_sl[sl].at[i]], sden[b],
                     add=True)

  def _wait_scatter(b):
    pltpu.make_async_copy(rows_bf[b], sh_numer.at[idxd_sl[0].at[0]],
                          srow[b]).wait()
    pltpu.make_async_copy(ex_bf[b], sh_denom.at[idxd_sl[0].at[0]],
                          sden[b]).wait()

  def _start_refill(sl, g_next):
    off = rbase + g_next
    pltpu.async_copy(src_hbm.at[pl.ds(off, NBUF)], idxs_sl[sl].at[...],
                     rsem[2 * sl])
    pltpu.async_copy(dst_hbm.at[pl.ds(off, NBUF)], idxd_sl[sl].at[...],
                     rsem[2 * sl + 1])

  def _wait_refill(sl):
    pltpu.make_async_copy(src_hbm.at[pl.ds(0, NBUF)], idxs_sl[sl].at[...],
                          rsem[2 * sl]).wait()
    pltpu.make_async_copy(dst_hbm.at[pl.ds(0, NBUF)], idxd_sl[sl].at[...],
                          rsem[2 * sl + 1]).wait()

  _start_gather(0, 0, 0)
  _start_gather(0, 1, 1)

  @pl.loop(0, NB_TILE, step=GSTEP)
  def _batch(g):
    for half in range(2):
      sl = half
      so = 1 - half
      gb = g + half * NBUF
      for i in range(NBUF):
        b = i
        _ex_compute(sl, i, ex_bf[b])
        _wait_gather(sl, i, b)
        n2 = (i + 2) % NBUF
        if half == 0 and i < 2:
          @pl.when(g > 0)
          def _():
            _wait_scatter(n2)
        else:
          _wait_scatter(n2)
        if i == 1:
          if half == 0:
            _start_refill(so, gb + NBUF)
          else:
            @pl.when(g + GSTEP < NB_TILE)
            def _():
              _start_refill(so, gb + NBUF)
        if i < 2:
          _start_gather(sl, i + 2, n2)
        elif i == 2:
          if half == 0:
            _wait_refill(so)
            _start_gather(so, 0, n2)
          else:
            @pl.when(g + GSTEP < NB_TILE)
            def _():
              _wait_refill(so)
              _start_gather(so, 0, n2)
        else:
          if half == 0:
            _start_gather(so, 1, n2)
          else:
            @pl.when(g + GSTEP < NB_TILE)
            def _():
              _start_gather(so, 1, n2)
        _scale(rows_bf[b], ex_bf[b])
        _start_scatter(sl, i, b)

  for i in range(2, NBUF):
    _wait_scatter(i)

  plsc.subcore_barrier()
  pltpu.sync_copy(sh_numer.at[pl.ds(r0, ROWS_PER_TILE)],
                  numer_hbm.at[c, pl.ds(r0, ROWS_PER_TILE)])
  pltpu.sync_copy(sh_denom.at[pl.ds(r0, ROWS_PER_TILE)],
                  denom_hbm.at[c, pl.ds(r0, ROWS_PER_TILE)])


def kernel(x, edge_index, W1s, W1d, a1s, a1d, b1, Wl1, bl1,
           W2s, W2d, a2s, a2d, b2, Wl2, bl2):
  src = edge_index[0].astype(jnp.int32)
  dst = edge_index[1].astype(jnp.int32)
  pad = jnp.full((EPAD - E,), N, jnp.int32)
  srcp = jnp.concatenate([src, pad]).reshape(NS * NB_TILE, B)
  dstp = jnp.concatenate([dst, pad]).reshape(NS * NB_TILE, B)

  xp = jnp.zeros((NPAD, D), jnp.float32).at[:N].set(x)
  zr = jnp.zeros((ROWS_PER_TILE, DH), jnp.float32)
  zv = jnp.zeros((ROWS_PER_TILE,), jnp.float32)

  a1s_v = a1s.reshape(1, D)
  a1d_v = a1d.reshape(1, D)
  a2s_v = a2s.reshape(1, D)
  a2d_v = a2d.reshape(1, D)

  sc_edge = _make_sc_edge_kernel()

  xs1, asrc1, ad1, skip1 = _tc_prep(
      xp, W1s, W1d, a1s_v, a1d_v, Wl1, bl1.reshape(1, D))
  m1 = _tc_m(asrc1)
  numer1, denom1 = sc_edge(
      xs1, asrc1.reshape(NPAD), ad1.reshape(NPAD), m1,
      srcp, dstp, zr, zv)

  xs2, asrc2, ad2, skip2 = _tc_mid(
      numer1, denom1[0].reshape(NPAD, 1), b1.reshape(1, D), skip1,
      W2s, W2d, a2s_v, a2d_v, Wl2, bl2.reshape(1, D))
  m2 = _tc_m(asrc2)
  numer2, denom2 = sc_edge(
      xs2, asrc2.reshape(NPAD), ad2.reshape(NPAD), m2,
      srcp, dstp, zr, zv)

  out = _tc_final(numer2, denom2[0].reshape(NPAD, 1), b2.reshape(1, D),
                  skip2)
  return out[:N]
